# baseline (speedup 1.0000x reference)
import functools

import jax
import jax.numpy as jnp
from jax.experimental import pallas as pl
from jax.experimental.pallas import tpu as pltpu


_PACK = 8
_LANES = 128


def _mlp_body(x_ref, w1_ref, b1_ref, w2_ref, b2_ref, o_ref):
    rows = x_ref.shape[0] // _LANES
    x = x_ref[...].reshape(rows, _LANES)
    h = jnp.dot(x, w1_ref[...], preferred_element_type=jnp.float32)
    h = jnp.maximum(h + b1_ref[...], 0.0)
    y = jnp.dot(h, w2_ref[...], preferred_element_type=jnp.float32)
    o_ref[...] = jnp.maximum(y + b2_ref[...], 0.0).reshape(x_ref.shape)


@functools.partial(jax.jit, static_argnames=("chunk",))
def _run(xf, w1b, b1b, w2b, b2b, chunk):
    n = xf.shape[0]
    grid = n // chunk
    return pl.pallas_call(
        _mlp_body,
        grid=(grid,),
        in_specs=[
            pl.BlockSpec((chunk,), lambda i: (i,)),
            pl.BlockSpec((_LANES, _LANES), lambda i: (0, 0)),
            pl.BlockSpec((1, _LANES), lambda i: (0, 0)),
            pl.BlockSpec((_LANES, _LANES), lambda i: (0, 0)),
            pl.BlockSpec((1, _LANES), lambda i: (0, 0)),
        ],
        out_specs=pl.BlockSpec((chunk,), lambda i: (i,)),
        out_shape=jax.ShapeDtypeStruct((n,), jnp.float32),
        compiler_params=pltpu.CompilerParams(
            dimension_semantics=("arbitrary",),
        ),
    )(xf, w1b, b1b, w2b, b2b)


def kernel(edge_attr, W1, b1, W2, b2):
    e, d = edge_attr.shape
    eye = jnp.eye(_PACK, dtype=jnp.float32)
    w1b = jnp.kron(eye, W1.astype(jnp.float32))
    w2b = jnp.kron(eye, W2.astype(jnp.float32))
    b1b = jnp.tile(b1.astype(jnp.float32), _PACK).reshape(1, _LANES)
    b2b = jnp.tile(b2.astype(jnp.float32), _PACK).reshape(1, _LANES)
    n = e * d
    chunk = next(c for c in (512000, 256000, 128000, 64000, 16000, 2048, 128)
                 if n % c == 0)
    xf = edge_attr.reshape(n)
    out = _run(xf, w1b, b1b, w2b, b2b, chunk=chunk)
    return out.reshape(e, d)


# speedup vs baseline: 16.5793x; 16.5793x over previous
import functools

import jax
import jax.numpy as jnp
from jax.experimental import pallas as pl
from jax.experimental.pallas import tpu as pltpu


def _mlp_body(x_ref, w1t_ref, b1_ref, w2t_ref, b2_ref, o_ref):
    x = x_ref[...]
    h = jnp.dot(w1t_ref[...], x, preferred_element_type=jnp.float32)
    h = jnp.maximum(h + b1_ref[...], 0.0)
    y = jnp.dot(w2t_ref[...], h, preferred_element_type=jnp.float32)
    o_ref[...] = jnp.maximum(y + b2_ref[...], 0.0)


@functools.partial(jax.jit, static_argnames=("block_cols",))
def _run(xt, w1t, b1c, w2t, b2c, block_cols):
    d, e = xt.shape
    grid = e // block_cols
    return pl.pallas_call(
        _mlp_body,
        grid=(grid,),
        in_specs=[
            pl.BlockSpec((d, block_cols), lambda i: (0, i)),
            pl.BlockSpec((d, d), lambda i: (0, 0)),
            pl.BlockSpec((d, 1), lambda i: (0, 0)),
            pl.BlockSpec((d, d), lambda i: (0, 0)),
            pl.BlockSpec((d, 1), lambda i: (0, 0)),
        ],
        out_specs=pl.BlockSpec((d, block_cols), lambda i: (0, i)),
        out_shape=jax.ShapeDtypeStruct((d, e), jnp.float32),
        compiler_params=pltpu.CompilerParams(
            dimension_semantics=("arbitrary",),
        ),
    )(xt, w1t, b1c, w2t, b2c)


def kernel(edge_attr, W1, b1, W2, b2):
    e, d = edge_attr.shape
    xt = edge_attr.T
    w1t = W1.astype(jnp.float32).T
    w2t = W2.astype(jnp.float32).T
    b1c = b1.astype(jnp.float32).reshape(d, 1)
    b2c = b2.astype(jnp.float32).reshape(d, 1)
    block_cols = next(bc for bc in (32000, 16000, 8000, 4000, 2000, 1000, 128)
                      if e % bc == 0)
    out_t = _run(xt, w1t, b1c, w2t, b2c, block_cols=block_cols)
    return out_t.T


# speedup vs baseline: 19.6695x; 1.1864x over previous
import functools

import jax
import jax.numpy as jnp
from jax.experimental import pallas as pl
from jax.experimental.pallas import tpu as pltpu


def _mlp_body(x_ref, w1t_ref, b1_ref, w2t_ref, b2_ref, o_ref):
    x = x_ref[...]
    h = jnp.dot(w1t_ref[...], x, preferred_element_type=jnp.float32)
    h = jnp.maximum(h + b1_ref[...], 0.0)
    y = jnp.dot(w2t_ref[...], h, preferred_element_type=jnp.float32)
    o_ref[...] = jnp.maximum(y + b2_ref[...], 0.0)


@functools.partial(jax.jit, static_argnames=("block_cols",))
def _run(xt, w1t, b1c, w2t, b2c, block_cols):
    d, e = xt.shape
    grid = e // block_cols
    return pl.pallas_call(
        _mlp_body,
        grid=(grid,),
        in_specs=[
            pl.BlockSpec((d, block_cols), lambda i: (0, i)),
            pl.BlockSpec((d, d), lambda i: (0, 0)),
            pl.BlockSpec((d, 1), lambda i: (0, 0)),
            pl.BlockSpec((d, d), lambda i: (0, 0)),
            pl.BlockSpec((d, 1), lambda i: (0, 0)),
        ],
        out_specs=pl.BlockSpec((d, block_cols), lambda i: (0, i)),
        out_shape=jax.ShapeDtypeStruct((d, e), jnp.float32),
        compiler_params=pltpu.CompilerParams(
            dimension_semantics=("arbitrary",),
        ),
    )(xt, w1t, b1c, w2t, b2c)


def kernel(edge_attr, W1, b1, W2, b2):
    e, d = edge_attr.shape
    xt = edge_attr.T
    w1t = W1.astype(jnp.float32).T
    w2t = W2.astype(jnp.float32).T
    b1c = b1.astype(jnp.float32).reshape(d, 1)
    b2c = b2.astype(jnp.float32).reshape(d, 1)
    block_cols = next(bc for bc in (64000, 32000, 16000, 8000, 2000, 1000, 128)
                      if e % bc == 0)
    out_t = _run(xt, w1t, b1c, w2t, b2c, block_cols=block_cols)
    return out_t.T


# speedup vs baseline: 20.5624x; 1.0454x over previous
import functools

import jax
import jax.numpy as jnp
from jax.experimental import pallas as pl
from jax.experimental.pallas import tpu as pltpu


def _mlp_body(x_ref, w1t_ref, b1_ref, w2t_ref, b2_ref, o_ref):
    x = x_ref[...]
    h = jnp.dot(w1t_ref[...], x, preferred_element_type=jnp.float32)
    h = jnp.maximum(h + b1_ref[...], 0.0)
    y = jnp.dot(w2t_ref[...], h, preferred_element_type=jnp.float32)
    o_ref[...] = jnp.maximum(y + b2_ref[...], 0.0)


@functools.partial(jax.jit, static_argnames=("block_cols",))
def _run(xt, w1t, b1c, w2t, b2c, block_cols):
    d, e = xt.shape
    grid = e // block_cols
    return pl.pallas_call(
        _mlp_body,
        grid=(grid,),
        in_specs=[
            pl.BlockSpec((d, block_cols), lambda i: (0, i)),
            pl.BlockSpec((d, d), lambda i: (0, 0)),
            pl.BlockSpec((d, 1), lambda i: (0, 0)),
            pl.BlockSpec((d, d), lambda i: (0, 0)),
            pl.BlockSpec((d, 1), lambda i: (0, 0)),
        ],
        out_specs=pl.BlockSpec((d, block_cols), lambda i: (0, i)),
        out_shape=jax.ShapeDtypeStruct((d, e), jnp.float32),
        compiler_params=pltpu.CompilerParams(
            dimension_semantics=("arbitrary",),
        ),
    )(xt, w1t, b1c, w2t, b2c)


def kernel(edge_attr, W1, b1, W2, b2):
    e, d = edge_attr.shape
    xt = edge_attr.T
    w1t = W1.astype(jnp.float32).T
    w2t = W2.astype(jnp.float32).T
    b1c = b1.astype(jnp.float32).reshape(d, 1)
    b2c = b2.astype(jnp.float32).reshape(d, 1)
    block_cols = next(bc for bc in (160000, 64000, 32000, 16000, 2000, 1000, 128)
                      if e % bc == 0)
    out_t = _run(xt, w1t, b1c, w2t, b2c, block_cols=block_cols)
    return out_t.T
